# baseline (speedup 1.0000x reference)
import jax
import jax.numpy as jnp
from jax import lax
from jax.experimental import pallas as pl
from jax.experimental.pallas import tpu as pltpu
from jax.experimental.pallas import tpu_sc as plsc

N = 100000
D = 128
K = 7
NSLOT = K - 1

BM = 1000
GRID = N // BM


def _mm_body(x_ref, w_ref, b_ref, base_ref, *t_refs):
    y = jnp.dot(
        x_ref[...],
        w_ref[...],
        preferred_element_type=jnp.float32,
    )
    base_ref[...] = y[:, :D] + b_ref[...]
    for k in range(NSLOT):
        c0 = D * (k + 1)
        t_refs[k][...] = y[:, c0:c0 + D]


def _tc_matmul(x, wcat, bias2d):
    outs = [jax.ShapeDtypeStruct((N, D), jnp.float32) for _ in range(1 + NSLOT)]
    return pl.pallas_call(
        _mm_body,
        grid=(GRID,),
        in_specs=[
            pl.BlockSpec((BM, D), lambda i: (i, 0)),
            pl.BlockSpec((D, K * D), lambda i: (0, 0)),
            pl.BlockSpec((1, D), lambda i: (0, 0)),
        ],
        out_specs=[pl.BlockSpec((BM, D), lambda i: (i, 0))] * (1 + NSLOT),
        out_shape=outs,
        compiler_params=pltpu.CompilerParams(
            dimension_semantics=("arbitrary",),
        ),
    )(x, wcat, bias2d)


NW = 32
PER_W = 3200
CH = 80
NT_FULL = PER_W // CH
NT_LAST = (N - (NW - 1) * PER_W) // CH
PAD_N = NW * PER_W


def _sc_body(t0, t1, t2, t3, t4, t5, base_hbm,
             ih0, ih1, ih2, ih3, ih4, ih5, wh0, wh1, wh2, wh3, wh4, wh5,
             out_hbm,
             i0, i1, i2, i3, i4, i5, w0s, w1s, w2s, w3s, w4s, w5s,
             g0, g1, g2, g3, g4, g5, acc_v, gsem):
    tabs = (t0, t1, t2, t3, t4, t5)
    idx_hbm = (ih0, ih1, ih2, ih3, ih4, ih5)
    w_hbm = (wh0, wh1, wh2, wh3, wh4, wh5)
    idxs = (i0, i1, i2, i3, i4, i5)
    wss = (w0s, w1s, w2s, w3s, w4s, w5s)
    gs = (g0, g1, g2, g3, g4, g5)
    wid = lax.axis_index("s") * 2 + lax.axis_index("c")
    w0 = pl.multiple_of(wid * PER_W, 8)
    for j in range(NSLOT):
        pltpu.sync_copy(idx_hbm[j].at[pl.ds(w0, PER_W)], idxs[j])
        pltpu.sync_copy(w_hbm[j].at[pl.ds(w0, PER_W)], wss[j].at[pl.ds(0, PER_W)])
    nt = jnp.where(wid == NW - 1, NT_LAST, NT_FULL)

    def chunk(t, carry):
        r0 = pl.multiple_of(t * CH, 8)
        b0 = pl.multiple_of(wid * PER_W + t * CH, 8)
        cps = [
            pltpu.async_copy(tabs[j].at[idxs[j].at[pl.ds(r0, CH)]], gs[j], gsem)
            for j in range(NSLOT)
        ]
        pltpu.sync_copy(base_hbm.at[pl.ds(b0, CH)], acc_v)
        for cp in cps:
            cp.wait()

        @plsc.parallel_loop(0, CH, 1, unroll=4)
        def row(i):
            r = r0 + i
            ws = [wss[j][pl.ds(r, 16)][0] for j in range(NSLOT)]
            for q in range(D // 16):
                sl = pl.ds(q * 16, 16)
                v = acc_v[i, sl]
                for j in range(NSLOT):
                    v = v + ws[j] * gs[j][i, sl]
                acc_v[i, sl] = v
        pltpu.sync_copy(acc_v, out_hbm.at[pl.ds(b0, CH)])
        return carry

    lax.fori_loop(0, nt, chunk, jnp.int32(0))


_sc_gather_accum = pl.kernel(
    _sc_body,
    out_type=jax.ShapeDtypeStruct((N, D), jnp.float32),
    mesh=plsc.VectorSubcoreMesh(core_axis_name="c", subcore_axis_name="s"),
    scratch_types=(
        [pltpu.VMEM((PER_W,), jnp.int32) for _ in range(NSLOT)]
        + [pltpu.VMEM((PER_W + 16,), jnp.float32) for _ in range(NSLOT)]
        + [pltpu.VMEM((CH, D), jnp.float32) for _ in range(NSLOT)]
        + [
            pltpu.VMEM((CH, D), jnp.float32),
            pltpu.SemaphoreType.DMA,
        ]
    ),
)


def kernel(x, neighbors_indices, neighbors_weights, weight, bias):
    wcat = weight.transpose(1, 0, 2).reshape(D, K * D)
    base, *tabs = _tc_matmul(x, wcat, bias.reshape(1, D))
    pad = PAD_N - N
    idx_cols = [jnp.pad(neighbors_indices[:, j], (0, pad)) for j in range(NSLOT)]
    w_cols = [jnp.pad(neighbors_weights[:, j], (0, pad)) for j in range(NSLOT)]
    return _sc_gather_accum(*tabs, base, *idx_cols, *w_cols)


# speedup vs baseline: 1.1365x; 1.1365x over previous
import jax
import jax.numpy as jnp
from jax import lax
from jax.experimental import pallas as pl
from jax.experimental.pallas import tpu as pltpu
from jax.experimental.pallas import tpu_sc as plsc

N = 100000
D = 128
K = 7
NSLOT = K - 1

BM = 1000
GRID = N // BM


def _mm_body(x_ref, w_ref, b_ref, base_ref, *t_refs):
    y = jnp.dot(
        x_ref[...],
        w_ref[...],
        preferred_element_type=jnp.float32,
    )
    base_ref[...] = y[:, :D] + b_ref[...]
    for k in range(NSLOT):
        c0 = D * (k + 1)
        t_refs[k][...] = y[:, c0:c0 + D]


def _tc_matmul(x, wcat, bias2d):
    outs = [jax.ShapeDtypeStruct((N, D), jnp.float32) for _ in range(1 + NSLOT)]
    return pl.pallas_call(
        _mm_body,
        grid=(GRID,),
        in_specs=[
            pl.BlockSpec((BM, D), lambda i: (i, 0)),
            pl.BlockSpec((D, K * D), lambda i: (0, 0)),
            pl.BlockSpec((1, D), lambda i: (0, 0)),
        ],
        out_specs=[pl.BlockSpec((BM, D), lambda i: (i, 0))] * (1 + NSLOT),
        out_shape=outs,
        compiler_params=pltpu.CompilerParams(
            dimension_semantics=("arbitrary",),
        ),
    )(x, wcat, bias2d)


NW = 32
PER_W = 3200
CH = 80
NT_FULL = PER_W // CH
NT_LAST = (N - (NW - 1) * PER_W) // CH
PAD_N = NW * PER_W


def _sc_body(t0, t1, t2, t3, t4, t5, base_hbm,
             ih0, ih1, ih2, ih3, ih4, ih5, wh0, wh1, wh2, wh3, wh4, wh5,
             out_hbm,
             i0, i1, i2, i3, i4, i5, w0s, w1s, w2s, w3s, w4s, w5s,
             g0, g1, g2, g3, g4, g5, acc_v, gsem):
    tabs = (t0, t1, t2, t3, t4, t5)
    idx_hbm = (ih0, ih1, ih2, ih3, ih4, ih5)
    w_hbm = (wh0, wh1, wh2, wh3, wh4, wh5)
    idxs = (i0, i1, i2, i3, i4, i5)
    wss = (w0s, w1s, w2s, w3s, w4s, w5s)
    gs = (g0, g1, g2, g3, g4, g5)
    wid = lax.axis_index("s") * 2 + lax.axis_index("c")
    w0 = pl.multiple_of(wid * PER_W, 8)
    for j in range(NSLOT):
        pltpu.sync_copy(idx_hbm[j].at[pl.ds(w0, PER_W)], idxs[j])
        pltpu.sync_copy(w_hbm[j].at[pl.ds(w0, PER_W)], wss[j].at[pl.ds(0, PER_W)])
    nt = jnp.where(wid == NW - 1, NT_LAST, NT_FULL)

    def chunk(t, carry):
        r0 = pl.multiple_of(t * CH, 8)
        b0 = pl.multiple_of(wid * PER_W + t * CH, 8)
        cps = [
            pltpu.async_copy(tabs[j].at[idxs[j].at[pl.ds(r0, CH)]], gs[j], gsem)
            for j in range(NSLOT)
        ]
        pltpu.sync_copy(base_hbm.at[pl.ds(b0, CH)], acc_v)
        for cp in cps:
            cp.wait()

        @plsc.parallel_loop(0, CH, 1, unroll=2)
        def row(i):
            r = r0 + i
            ws = [wss[j][pl.ds(r, 16)][0] for j in range(NSLOT)]
            for q in range(D // 16):
                sl = pl.ds(q * 16, 16)
                v = acc_v[i, sl]
                for j in range(NSLOT):
                    v = v + ws[j] * gs[j][i, sl]
                acc_v[i, sl] = v
        pltpu.sync_copy(acc_v, out_hbm.at[pl.ds(b0, CH)])
        return carry

    lax.fori_loop(0, nt, chunk, jnp.int32(0))


_sc_gather_accum = pl.kernel(
    _sc_body,
    out_type=jax.ShapeDtypeStruct((N, D), jnp.float32),
    mesh=plsc.VectorSubcoreMesh(core_axis_name="c", subcore_axis_name="s"),
    scratch_types=(
        [pltpu.VMEM((PER_W,), jnp.int32) for _ in range(NSLOT)]
        + [pltpu.VMEM((PER_W + 16,), jnp.float32) for _ in range(NSLOT)]
        + [pltpu.VMEM((CH, D), jnp.float32) for _ in range(NSLOT)]
        + [
            pltpu.VMEM((CH, D), jnp.float32),
            pltpu.SemaphoreType.DMA,
        ]
    ),
)


def kernel(x, neighbors_indices, neighbors_weights, weight, bias):
    wcat = weight.transpose(1, 0, 2).reshape(D, K * D)
    base, *tabs = _tc_matmul(x, wcat, bias.reshape(1, D))
    pad = PAD_N - N
    idx_cols = [jnp.pad(neighbors_indices[:, j], (0, pad)) for j in range(NSLOT)]
    w_cols = [jnp.pad(neighbors_weights[:, j], (0, pad)) for j in range(NSLOT)]
    return _sc_gather_accum(*tabs, base, *idx_cols, *w_cols)


# speedup vs baseline: 1.1694x; 1.0289x over previous
import jax
import jax.numpy as jnp
from jax import lax
from jax.experimental import pallas as pl
from jax.experimental.pallas import tpu as pltpu
from jax.experimental.pallas import tpu_sc as plsc

N = 100000
D = 128
K = 7
NSLOT = K - 1

BM = 1000
GRID = N // BM


def _mm_body(x_ref, w_ref, b_ref, base_ref, *t_refs):
    y = jnp.dot(
        x_ref[...],
        w_ref[...],
        preferred_element_type=jnp.float32,
    )
    base_ref[...] = y[:, :D] + b_ref[...]
    for k in range(NSLOT):
        c0 = D * (k + 1)
        t_refs[k][...] = y[:, c0:c0 + D]


def _tc_matmul(x, wcat, bias2d):
    outs = [jax.ShapeDtypeStruct((N, D), jnp.float32) for _ in range(1 + NSLOT)]
    return pl.pallas_call(
        _mm_body,
        grid=(GRID,),
        in_specs=[
            pl.BlockSpec((BM, D), lambda i: (i, 0)),
            pl.BlockSpec((D, K * D), lambda i: (0, 0)),
            pl.BlockSpec((1, D), lambda i: (0, 0)),
        ],
        out_specs=[pl.BlockSpec((BM, D), lambda i: (i, 0))] * (1 + NSLOT),
        out_shape=outs,
        compiler_params=pltpu.CompilerParams(
            dimension_semantics=("arbitrary",),
        ),
    )(x, wcat, bias2d)


NW = 32
PER_W = 3200
CH = 80
NT_FULL = PER_W // CH
NT_LAST = (N - (NW - 1) * PER_W) // CH
PAD_N = NW * PER_W


def _sc_body(t0, t1, t2, t3, t4, t5, base_hbm,
             ih0, ih1, ih2, ih3, ih4, ih5, wh0, wh1, wh2, wh3, wh4, wh5,
             out_hbm,
             i0, i1, i2, i3, i4, i5, w0s, w1s, w2s, w3s, w4s, w5s,
             g0, g1, g2, g3, g4, g5, acc_v, gsem, bsem, osem0, osem1):
    tabs = (t0, t1, t2, t3, t4, t5)
    idx_hbm = (ih0, ih1, ih2, ih3, ih4, ih5)
    w_hbm = (wh0, wh1, wh2, wh3, wh4, wh5)
    idxs = (i0, i1, i2, i3, i4, i5)
    wss = (w0s, w1s, w2s, w3s, w4s, w5s)
    gs = (g0, g1, g2, g3, g4, g5)
    wid = lax.axis_index("s") * 2 + lax.axis_index("c")
    w0 = pl.multiple_of(wid * PER_W, 8)
    for j in range(NSLOT):
        pltpu.sync_copy(idx_hbm[j].at[pl.ds(w0, PER_W)], idxs[j])
        pltpu.sync_copy(w_hbm[j].at[pl.ds(w0, PER_W)], wss[j].at[pl.ds(0, PER_W)])
    nt = jnp.where(wid == NW - 1, NT_LAST, NT_FULL)

    def out_copy(c):
        bc = pl.multiple_of(wid * PER_W + c * CH, 8)
        sbuf = c & 1
        return pltpu.make_async_copy(
            acc_v.at[pl.ds(pl.multiple_of(sbuf * CH, 8), CH)],
            out_hbm.at[pl.ds(bc, CH)], osem0)

    def chunk(t, carry):
        buf = t & 1
        ro = pl.multiple_of(buf * CH, 8)
        r0 = pl.multiple_of(t * CH, 8)
        b0 = pl.multiple_of(wid * PER_W + t * CH, 8)
        cps = [
            pltpu.async_copy(tabs[j].at[idxs[j].at[pl.ds(r0, CH)]], gs[j], gsem)
            for j in range(NSLOT)
        ]
        bcp = pltpu.make_async_copy(
            base_hbm.at[pl.ds(b0, CH)], acc_v.at[pl.ds(ro, CH)], bsem)
        bcp.start()
        bcp.wait()
        for cp in cps:
            cp.wait()

        @plsc.parallel_loop(0, CH, 1, unroll=2)
        def row(i):
            r = r0 + i
            ws = [wss[j][pl.ds(r, 16)][0] for j in range(NSLOT)]
            for q in range(D // 16):
                sl = pl.ds(q * 16, 16)
                v = acc_v[ro + i, sl]
                for j in range(NSLOT):
                    v = v + ws[j] * gs[j][ro + i, sl]
                acc_v[ro + i, sl] = v

        pl.when(t >= 1)(lambda: out_copy(t - 1).wait())
        out_copy(t).start()
        return carry

    lax.fori_loop(0, nt, chunk, jnp.int32(0))
    out_copy(nt - 1).wait()


_sc_gather_accum = pl.kernel(
    _sc_body,
    out_type=jax.ShapeDtypeStruct((N, D), jnp.float32),
    mesh=plsc.VectorSubcoreMesh(core_axis_name="c", subcore_axis_name="s"),
    scratch_types=(
        [pltpu.VMEM((PER_W,), jnp.int32) for _ in range(NSLOT)]
        + [pltpu.VMEM((PER_W + 16,), jnp.float32) for _ in range(NSLOT)]
        + [pltpu.VMEM((CH, D), jnp.float32) for _ in range(NSLOT)]
        + [
            pltpu.VMEM((2 * CH, D), jnp.float32),
            pltpu.SemaphoreType.DMA,
            pltpu.SemaphoreType.DMA,
            pltpu.SemaphoreType.DMA,
            pltpu.SemaphoreType.DMA,
        ]
    ),
)


def kernel(x, neighbors_indices, neighbors_weights, weight, bias):
    wcat = weight.transpose(1, 0, 2).reshape(D, K * D)
    base, *tabs = _tc_matmul(x, wcat, bias.reshape(1, D))
    pad = PAD_N - N
    idx_cols = [jnp.pad(neighbors_indices[:, j], (0, pad)) for j in range(NSLOT)]
    w_cols = [jnp.pad(neighbors_weights[:, j], (0, pad)) for j in range(NSLOT)]
    return _sc_gather_accum(*tabs, base, *idx_cols, *w_cols)
